# Initial kernel scaffold; baseline (speedup 1.0000x reference)
import functools

import jax
import jax.numpy as jnp
from jax import lax
from jax.experimental import pallas as pl
from jax.experimental.pallas import tpu as pltpu
from jax.experimental.pallas import tpu_sc as plsc

F = 26
V = 100000
D = 32
B = 16384
OUT = 128

NC, NS, LANES = 2, 16, 16
NW = NC * NS
BF = B * F
PER_W = BF // NW
CH = 128
N_CH = PER_W // CH


def _gather_sc(x_flat, offs, table_flat):
    mesh = plsc.VectorSubcoreMesh(core_axis_name="c", subcore_axis_name="s")

    @functools.partial(
        pl.kernel,
        mesh=mesh,
        out_type=jax.ShapeDtypeStruct((BF, D), jnp.float32),
        scratch_types=[
            pltpu.VMEM((PER_W,), jnp.int32),
            pltpu.VMEM((PER_W,), jnp.int32),
            pltpu.VMEM((CH, D), jnp.float32),
            pltpu.SemaphoreType.DMA,
        ],
    )
    def k(x_hbm, offs_hbm, tbl_hbm, out_hbm, idx_v, offs_v, rows_v, sem):
        wid = lax.axis_index("s") * NC + lax.axis_index("c")
        base = wid * PER_W
        pltpu.sync_copy(x_hbm.at[pl.ds(base, PER_W)], idx_v)
        pltpu.sync_copy(offs_hbm, offs_v)

        def add_body(i, carry):
            s = pl.ds(i * LANES, LANES)
            idx_v[s] = idx_v[s] + offs_v[s]
            return carry

        lax.fori_loop(0, PER_W // LANES, add_body, 0)

        def gather_body(j, carry):
            pltpu.async_copy(
                tbl_hbm.at[idx_v.at[pl.ds(j * CH, CH)]], rows_v, sem
            ).wait()
            pltpu.sync_copy(rows_v, out_hbm.at[pl.ds(base + j * CH, CH)])
            return carry

        lax.fori_loop(0, N_CH, gather_body, 0)

    return k(x_flat, offs, table_flat)


def _matmul_tc(a, w):
    BM = 1024

    def body(a_ref, w_ref, o_ref):
        o_ref[...] = jnp.dot(a_ref[...], w_ref[...],
                             preferred_element_type=jnp.float32)

    return pl.pallas_call(
        body,
        grid=(B // BM,),
        in_specs=[
            pl.BlockSpec((BM, F * D), lambda i: (i, 0)),
            pl.BlockSpec((F * D, OUT), lambda i: (0, 0)),
        ],
        out_specs=pl.BlockSpec((BM, OUT), lambda i: (i, 0)),
        out_shape=jax.ShapeDtypeStruct((B, OUT), jnp.float32),
    )(a, w)


def kernel(x, tables, W):
    table_flat = tables.reshape(F * V, D)
    x_flat = x.reshape(BF)
    offs = jnp.tile(jnp.arange(F, dtype=jnp.int32) * V, PER_W // F)
    gathered = _gather_sc(x_flat, offs, table_flat)
    return _matmul_tc(gathered.reshape(B, F * D), W)


# speedup vs baseline: 7.6882x; 7.6882x over previous
import functools

import jax
import jax.numpy as jnp
from jax import lax
from jax.experimental import pallas as pl
from jax.experimental.pallas import tpu as pltpu
from jax.experimental.pallas import tpu_sc as plsc

F = 26
V = 100000
D = 32
B = 16384
OUT = 128

NC, NS, LANES = 2, 16, 16
NW = NC * NS
BF = B * F
PER_W = BF // NW
CH = 128
N_CH = PER_W // CH


def _gather_sc(x_flat, offs, table_flat):
    mesh = plsc.VectorSubcoreMesh(core_axis_name="c", subcore_axis_name="s")

    @functools.partial(
        pl.kernel,
        mesh=mesh,
        out_type=jax.ShapeDtypeStruct((BF, D), jnp.float32),
        compiler_params=pltpu.CompilerParams(use_tc_tiling_on_sc=False),
        scratch_types=[
            pltpu.VMEM((PER_W,), jnp.int32),
            pltpu.VMEM((PER_W,), jnp.int32),
            pltpu.VMEM((CH, D), jnp.float32),
            pltpu.SemaphoreType.DMA,
        ],
    )
    def k(x_hbm, offs_hbm, tbl_hbm, out_hbm, idx_v, offs_v, rows_v, sem):
        wid = lax.axis_index("s") * NC + lax.axis_index("c")
        base = wid * PER_W
        pltpu.sync_copy(x_hbm.at[pl.ds(base, PER_W)], idx_v)
        pltpu.sync_copy(offs_hbm, offs_v)

        def add_body(i, carry):
            s = pl.ds(i * LANES, LANES)
            idx_v[s] = idx_v[s] + offs_v[s]
            return carry

        lax.fori_loop(0, PER_W // LANES, add_body, 0)

        def gather_body(j, carry):
            pltpu.async_copy(
                tbl_hbm.at[idx_v.at[pl.ds(j * CH, CH)]], rows_v, sem
            ).wait()
            pltpu.sync_copy(rows_v, out_hbm.at[pl.ds(base + j * CH, CH)])
            return carry

        lax.fori_loop(0, N_CH, gather_body, 0)

    return k(x_flat, offs, table_flat)


def _matmul_tc(a, w):
    BM = 1024

    def body(a_ref, w_ref, o_ref):
        o_ref[...] = jnp.dot(a_ref[...], w_ref[...],
                             preferred_element_type=jnp.float32)

    return pl.pallas_call(
        body,
        grid=(B // BM,),
        in_specs=[
            pl.BlockSpec((BM, F * D), lambda i: (i, 0)),
            pl.BlockSpec((F * D, OUT), lambda i: (0, 0)),
        ],
        out_specs=pl.BlockSpec((BM, OUT), lambda i: (i, 0)),
        out_shape=jax.ShapeDtypeStruct((B, OUT), jnp.float32),
    )(a, w)


def kernel(x, tables, W):
    table_flat = tables.reshape(F * V, D)
    x_flat = x.reshape(BF)
    offs = jnp.tile(jnp.arange(F, dtype=jnp.int32) * V, PER_W // F)
    gathered = _gather_sc(x_flat, offs, table_flat)
    return _matmul_tc(gathered.reshape(B, F * D), W)
